# baseline (speedup 1.0000x reference)
import jax
import jax.numpy as jnp
from jax import lax
from jax.experimental import pallas as pl
from jax.experimental.pallas import tpu as pltpu
from jax.experimental.pallas import tpu_sc as plsc

N_NODES = 10000
N_PAD = 10240
D = 128
N_EDGES = 320000
NC, NS = 2, 16
CHUNKS, CHUNK = 125, 80
GROUPS = 5
GCHUNKS = CHUNKS // GROUPS
ROWS_PER_TILE = N_PAD // NS
BLK = 1000


def _mesh():
    return plsc.VectorSubcoreMesh(
        core_axis_name="c", subcore_axis_name="s", num_cores=NC, num_subcores=NS
    )


def _zero_vmem_2d(ref, rows):
    @pl.loop(0, rows)
    def _(r):
        @pl.loop(0, D // 16)
        def _(c):
            ref[r, pl.ds(c * 16, 16)] = jnp.zeros((16,), jnp.float32)


def _deg_body(ei_hbm, deg_out, dsti_v, ones_v, zrow_v, dsem, deg_sh):
    cid = lax.axis_index("c")
    sid = lax.axis_index("s")
    base = pl.multiple_of(sid * ROWS_PER_TILE, ROWS_PER_TILE)

    @pl.loop(0, ROWS_PER_TILE // 16)
    def _(i):
        zrow_v[pl.ds(i * 16, 16)] = jnp.zeros((16,), jnp.float32)

    pltpu.sync_copy(zrow_v, deg_sh.at[pl.ds(base, ROWS_PER_TILE)])
    for g in range(GROUPS):
        pltpu.sync_copy(
            ei_hbm.at[1, cid, sid, g], dsti_v.at[pl.ds(g * GCHUNKS, GCHUNKS)]
        )

    @pl.loop(0, CHUNK // 16)
    def _(i):
        ones_v[pl.ds(i * 16, 16)] = jnp.full((16,), 1.0, jnp.float32)

    plsc.subcore_barrier()

    W_DEG = 8

    def dscat(j):
        pltpu.async_copy(ones_v, deg_sh.at[dsti_v.at[j]], dsem, add=True)

    def dwait(j):
        pltpu.make_async_copy(ones_v, deg_sh.at[dsti_v.at[j]], dsem).wait()

    for j in range(W_DEG):
        dscat(j)

    @pl.loop(W_DEG, CHUNKS)
    def _(j):
        dscat(j)
        dwait(j - W_DEG)

    @pl.loop(CHUNKS - W_DEG, CHUNKS)
    def _(j):
        dwait(j)

    plsc.subcore_barrier()
    pltpu.sync_copy(
        deg_sh.at[pl.ds(base, ROWS_PER_TILE)],
        deg_out.at[cid, pl.ds(base, ROWS_PER_TILE)],
    )


def _sc_deg(ei6):
    fn = pl.kernel(
        _deg_body,
        out_type=jax.ShapeDtypeStruct((NC, N_PAD), jnp.float32),
        mesh=_mesh(),
        scratch_types=[
            pltpu.VMEM((CHUNKS, CHUNK), jnp.int32),
            pltpu.VMEM((CHUNK,), jnp.float32),
            pltpu.VMEM((ROWS_PER_TILE,), jnp.float32),
            pltpu.SemaphoreType.DMA,
            pltpu.VMEM_SHARED((N_PAD,), jnp.float32),
        ],
    )
    return fn(ei6)


def _prop_body(h2_hbm, ei_hbm, acc_out, srci_v, dsti_v, rows, gsems, ssems, acc_sh):
    cid = lax.axis_index("c")
    sid = lax.axis_index("s")
    base = pl.multiple_of(sid * ROWS_PER_TILE, ROWS_PER_TILE)

    _zero_vmem_2d(rows.at[0], CHUNK)

    @pl.loop(0, ROWS_PER_TILE // CHUNK)
    def _(i):
        pltpu.sync_copy(rows.at[0], acc_sh.at[pl.ds(base + i * CHUNK, CHUNK), :])

    plsc.subcore_barrier()

    def gather(j, k):
        pltpu.async_copy(h2_hbm.at[srci_v.at[j]], rows.at[k], gsems.at[k])

    def gather_wait(j, k):
        pltpu.make_async_copy(h2_hbm.at[srci_v.at[j]], rows.at[k], gsems.at[k]).wait()

    def scatter(j, k):
        pltpu.async_copy(rows.at[k], acc_sh.at[dsti_v.at[j]], ssems.at[k], add=True)

    def scatter_wait(j, k):
        pltpu.make_async_copy(rows.at[k], acc_sh.at[dsti_v.at[j]], ssems.at[k]).wait()

    @pl.loop(0, GROUPS)
    def _(g):
        pltpu.sync_copy(ei_hbm.at[0, cid, sid, g], srci_v)
        pltpu.sync_copy(ei_hbm.at[1, cid, sid, g], dsti_v)
        gather(0, 0)
        gather(1, 1)
        gather(2, 2)

        @pl.loop(0, GCHUNKS // 5)
        def _(tt):
            jb = 4 * tt

            def step(o):
                j = jb + o
                gather_wait(j, o)
                scatter(j, o)

                @pl.when(j >= 1)
                def _():
                    scatter_wait(j - 1, (o + 3) % 4)

                gather(j + 3, (o + 3) % 4)

            step(0)
            step(1)
            step(2)
            step(3)

        jb = GCHUNKS - 5
        gather_wait(jb, jb % 4)
        scatter(jb, jb % 4)
        scatter_wait(jb - 1, (jb + 3) % 4)
        gather(jb + 3, (jb + 3) % 4)
        gather_wait(jb + 1, (jb + 1) % 4)
        scatter(jb + 1, (jb + 1) % 4)
        scatter_wait(jb, jb % 4)
        gather(jb + 4, (jb + 4) % 4)
        gather_wait(jb + 2, (jb + 2) % 4)
        scatter(jb + 2, (jb + 2) % 4)
        scatter_wait(jb + 1, (jb + 1) % 4)
        gather_wait(jb + 3, (jb + 3) % 4)
        scatter(jb + 3, (jb + 3) % 4)
        scatter_wait(jb + 2, (jb + 2) % 4)
        gather_wait(jb + 4, (jb + 4) % 4)
        scatter(jb + 4, (jb + 4) % 4)
        scatter_wait(jb + 3, (jb + 3) % 4)
        scatter_wait(jb + 4, (jb + 4) % 4)

    plsc.subcore_barrier()
    pltpu.sync_copy(
        acc_sh.at[pl.ds(base, ROWS_PER_TILE), :],
        acc_out.at[cid, pl.ds(base, ROWS_PER_TILE), :],
    )


def _sc_prop(h2, ei6):
    fn = pl.kernel(
        _prop_body,
        out_type=jax.ShapeDtypeStruct((NC, N_PAD, D), jnp.float32),
        mesh=_mesh(),
        scratch_types=[
            pltpu.VMEM((GCHUNKS, CHUNK), jnp.int32),
            pltpu.VMEM((GCHUNKS, CHUNK), jnp.int32),
            pltpu.VMEM((4, CHUNK, D), jnp.float32),
            pltpu.SemaphoreType.DMA((4,)),
            pltpu.SemaphoreType.DMA((4,)),
            pltpu.VMEM_SHARED((N_PAD, D), jnp.float32),
        ],
    )
    return fn(h2, ei6)


def _xform_body(x_ref, w_ref, dp_ref, h2_ref, dis_ref):
    deg = dp_ref[0] + dp_ref[1] + 1.0
    dis = lax.rsqrt(deg)
    dis_ref[...] = dis
    h = jnp.dot(x_ref[...], w_ref[...], preferred_element_type=jnp.float32)
    h2_ref[...] = h * dis


def _tc_xform(x, W, dp):
    return pl.pallas_call(
        _xform_body,
        grid=(N_NODES // BLK,),
        in_specs=[
            pl.BlockSpec((BLK, D), lambda i: (i, 0)),
            pl.BlockSpec((D, D), lambda i: (0, 0)),
            pl.BlockSpec((NC, BLK, 1), lambda i: (0, i, 0)),
        ],
        out_specs=[
            pl.BlockSpec((BLK, D), lambda i: (i, 0)),
            pl.BlockSpec((BLK, 1), lambda i: (i, 0)),
        ],
        out_shape=[
            jax.ShapeDtypeStruct((N_NODES, D), jnp.float32),
            jax.ShapeDtypeStruct((N_NODES, 1), jnp.float32),
        ],
    )(x, W, dp)


def _final_body(acc_ref, h2_ref, dis_ref, b_ref, out_ref):
    s = acc_ref[0] + acc_ref[1] + h2_ref[...]
    out_ref[...] = jnp.maximum(s * dis_ref[...] + b_ref[...], 0.0)


def _tc_final(acc, h2, dis, b2):
    return pl.pallas_call(
        _final_body,
        grid=(N_NODES // BLK,),
        in_specs=[
            pl.BlockSpec((NC, BLK, D), lambda i: (0, i, 0)),
            pl.BlockSpec((BLK, D), lambda i: (i, 0)),
            pl.BlockSpec((BLK, 1), lambda i: (i, 0)),
            pl.BlockSpec((1, D), lambda i: (0, 0)),
        ],
        out_specs=pl.BlockSpec((BLK, D), lambda i: (i, 0)),
        out_shape=jax.ShapeDtypeStruct((N_NODES, D), jnp.float32),
    )(acc, h2, dis, b2)


@jax.jit
def _impl(x, edge_index, W, b):
    ei6 = edge_index.astype(jnp.int32).reshape(2, NC, NS, GROUPS, GCHUNKS, CHUNK)

    deg_parts = _sc_deg(ei6)
    h2, dis = _tc_xform(x, W, deg_parts[:, :, None])
    acc = _sc_prop(h2, ei6)
    return _tc_final(acc, h2, dis, b.reshape(1, D))


def kernel(x, edge_index, W, b):
    return _impl(x, edge_index, W, b)


# speedup vs baseline: 1.0346x; 1.0346x over previous
import jax
import jax.numpy as jnp
from jax import lax
from jax.experimental import pallas as pl
from jax.experimental.pallas import tpu as pltpu
from jax.experimental.pallas import tpu_sc as plsc

N_NODES = 10000
N_PAD = 10240
D = 128
N_EDGES = 320000
NC, NS = 2, 16
CHUNKS, CHUNK = 125, 80
GROUPS = 5
GCHUNKS = CHUNKS // GROUPS
ROWS_PER_TILE = N_PAD // NS
BLK = 1000


def _mesh():
    return plsc.VectorSubcoreMesh(
        core_axis_name="c", subcore_axis_name="s", num_cores=NC, num_subcores=NS
    )


def _zero_vmem_2d(ref, rows):
    @pl.loop(0, rows)
    def _(r):
        @pl.loop(0, D // 16)
        def _(c):
            ref[r, pl.ds(c * 16, 16)] = jnp.zeros((16,), jnp.float32)


def _deg_body(ei_hbm, deg_out, dsti_v, ones_v, zrow_v, dsem, deg_sh):
    cid = lax.axis_index("c")
    sid = lax.axis_index("s")
    base = pl.multiple_of(sid * ROWS_PER_TILE, ROWS_PER_TILE)

    @pl.loop(0, ROWS_PER_TILE // 16)
    def _(i):
        zrow_v[pl.ds(i * 16, 16)] = jnp.zeros((16,), jnp.float32)

    pltpu.sync_copy(zrow_v, deg_sh.at[pl.ds(base, ROWS_PER_TILE)])
    for g in range(GROUPS):
        pltpu.sync_copy(
            ei_hbm.at[1, cid, sid, g], dsti_v.at[pl.ds(g * GCHUNKS, GCHUNKS)]
        )

    @pl.loop(0, CHUNK // 16)
    def _(i):
        ones_v[pl.ds(i * 16, 16)] = jnp.full((16,), 1.0, jnp.float32)

    plsc.subcore_barrier()

    W_DEG = 8

    def dscat(j):
        pltpu.async_copy(ones_v, deg_sh.at[dsti_v.at[j]], dsem, add=True)

    def dwait(j):
        pltpu.make_async_copy(ones_v, deg_sh.at[dsti_v.at[j]], dsem).wait()

    for j in range(W_DEG):
        dscat(j)

    @pl.loop(W_DEG, CHUNKS)
    def _(j):
        dscat(j)
        dwait(j - W_DEG)

    @pl.loop(CHUNKS - W_DEG, CHUNKS)
    def _(j):
        dwait(j)

    plsc.subcore_barrier()
    pltpu.sync_copy(
        deg_sh.at[pl.ds(base, ROWS_PER_TILE)],
        deg_out.at[cid, pl.ds(base, ROWS_PER_TILE)],
    )


def _sc_deg(ei6):
    fn = pl.kernel(
        _deg_body,
        out_type=jax.ShapeDtypeStruct((NC, N_PAD), jnp.float32),
        mesh=_mesh(),
        scratch_types=[
            pltpu.VMEM((CHUNKS, CHUNK), jnp.int32),
            pltpu.VMEM((CHUNK,), jnp.float32),
            pltpu.VMEM((ROWS_PER_TILE,), jnp.float32),
            pltpu.SemaphoreType.DMA,
            pltpu.VMEM_SHARED((N_PAD,), jnp.float32),
        ],
    )
    return fn(ei6)


def _prop_body(h2_hbm, ei_hbm, acc_out, srci_v, dsti_v, rows, gsems, acc_sh):
    cid = lax.axis_index("c")
    sid = lax.axis_index("s")
    base = pl.multiple_of(sid * ROWS_PER_TILE, ROWS_PER_TILE)

    _zero_vmem_2d(rows.at[0], CHUNK)

    @pl.loop(0, ROWS_PER_TILE // CHUNK)
    def _(i):
        pltpu.sync_copy(rows.at[0], acc_sh.at[pl.ds(base + i * CHUNK, CHUNK), :])

    plsc.subcore_barrier()

    def gather(j, k):
        pltpu.async_copy(h2_hbm.at[srci_v.at[j]], rows.at[k], gsems.at[k])

    def gather_wait(j, k):
        pltpu.make_async_copy(h2_hbm.at[srci_v.at[j]], rows.at[k], gsems.at[k]).wait()

    def scatter(j, k):
        pltpu.sync_copy(rows.at[k], acc_sh.at[dsti_v.at[j]], add=True)

    @pl.loop(0, GROUPS)
    def _(g):
        pltpu.sync_copy(ei_hbm.at[0, cid, sid, g], srci_v)
        pltpu.sync_copy(ei_hbm.at[1, cid, sid, g], dsti_v)
        gather(0, 0)
        gather(1, 1)
        gather(2, 2)

        @pl.loop(0, GCHUNKS // 5)
        def _(tt):
            jb = 4 * tt

            def step(o):
                j = jb + o
                gather(j + 3, (o + 3) % 4)
                gather_wait(j, o)
                scatter(j, o)

            step(0)
            step(1)
            step(2)
            step(3)

        jb = GCHUNKS - 5
        gather(jb + 3, (jb + 3) % 4)
        gather_wait(jb, jb % 4)
        scatter(jb, jb % 4)
        gather(jb + 4, (jb + 4) % 4)
        gather_wait(jb + 1, (jb + 1) % 4)
        scatter(jb + 1, (jb + 1) % 4)
        gather_wait(jb + 2, (jb + 2) % 4)
        scatter(jb + 2, (jb + 2) % 4)
        gather_wait(jb + 3, (jb + 3) % 4)
        scatter(jb + 3, (jb + 3) % 4)
        gather_wait(jb + 4, (jb + 4) % 4)
        scatter(jb + 4, (jb + 4) % 4)

    plsc.subcore_barrier()
    pltpu.sync_copy(
        acc_sh.at[pl.ds(base, ROWS_PER_TILE), :],
        acc_out.at[cid, pl.ds(base, ROWS_PER_TILE), :],
    )


def _sc_prop(h2, ei6):
    fn = pl.kernel(
        _prop_body,
        out_type=jax.ShapeDtypeStruct((NC, N_PAD, D), jnp.float32),
        mesh=_mesh(),
        scratch_types=[
            pltpu.VMEM((GCHUNKS, CHUNK), jnp.int32),
            pltpu.VMEM((GCHUNKS, CHUNK), jnp.int32),
            pltpu.VMEM((4, CHUNK, D), jnp.float32),
            pltpu.SemaphoreType.DMA((4,)),
            pltpu.VMEM_SHARED((N_PAD, D), jnp.float32),
        ],
    )
    return fn(h2, ei6)


def _xform_body(x_ref, w_ref, dp_ref, h2_ref, dis_ref):
    deg = dp_ref[0] + dp_ref[1] + 1.0
    dis = lax.rsqrt(deg)
    dis_ref[...] = dis
    h = jnp.dot(x_ref[...], w_ref[...], preferred_element_type=jnp.float32)
    h2_ref[...] = h * dis


def _tc_xform(x, W, dp):
    return pl.pallas_call(
        _xform_body,
        grid=(N_NODES // BLK,),
        in_specs=[
            pl.BlockSpec((BLK, D), lambda i: (i, 0)),
            pl.BlockSpec((D, D), lambda i: (0, 0)),
            pl.BlockSpec((NC, BLK, 1), lambda i: (0, i, 0)),
        ],
        out_specs=[
            pl.BlockSpec((BLK, D), lambda i: (i, 0)),
            pl.BlockSpec((BLK, 1), lambda i: (i, 0)),
        ],
        out_shape=[
            jax.ShapeDtypeStruct((N_NODES, D), jnp.float32),
            jax.ShapeDtypeStruct((N_NODES, 1), jnp.float32),
        ],
    )(x, W, dp)


def _final_body(acc_ref, h2_ref, dis_ref, b_ref, out_ref):
    s = acc_ref[0] + acc_ref[1] + h2_ref[...]
    out_ref[...] = jnp.maximum(s * dis_ref[...] + b_ref[...], 0.0)


def _tc_final(acc, h2, dis, b2):
    return pl.pallas_call(
        _final_body,
        grid=(N_NODES // BLK,),
        in_specs=[
            pl.BlockSpec((NC, BLK, D), lambda i: (0, i, 0)),
            pl.BlockSpec((BLK, D), lambda i: (i, 0)),
            pl.BlockSpec((BLK, 1), lambda i: (i, 0)),
            pl.BlockSpec((1, D), lambda i: (0, 0)),
        ],
        out_specs=pl.BlockSpec((BLK, D), lambda i: (i, 0)),
        out_shape=jax.ShapeDtypeStruct((N_NODES, D), jnp.float32),
    )(acc, h2, dis, b2)


@jax.jit
def _impl(x, edge_index, W, b):
    ei6 = edge_index.astype(jnp.int32).reshape(2, NC, NS, GROUPS, GCHUNKS, CHUNK)

    deg_parts = _sc_deg(ei6)
    h2, dis = _tc_xform(x, W, deg_parts[:, :, None])
    acc = _sc_prop(h2, ei6)
    return _tc_final(acc, h2, dis, b.reshape(1, D))


def kernel(x, edge_index, W, b):
    return _impl(x, edge_index, W, b)
